# Initial kernel scaffold; baseline (speedup 1.0000x reference)
import jax
import jax.numpy as jnp
from jax.experimental import pallas as pl


def kernel(seq, tok_table, pos_table):
    raise NotImplementedError("write your pallas kernel here")


# speedup vs baseline: 1.2783x; 1.2783x over previous
import functools

import jax
import jax.numpy as jnp
from jax import lax
from jax.experimental import pallas as pl
from jax.experimental.pallas import tpu as pltpu
from jax.experimental.pallas import tpu_sc as plsc

_VOCAB = 100000
_MAX_LEN = 2048
_EMBED_DIM = 128
_BATCH = 4
_B = _BATCH * _MAX_LEN
_NC = 2
_NS = 16
_NW = _NC * _NS
_BPW = _B // _NW
_L = 16


def _embed_body(seq_hbm, tok_hbm, pos_hbm, out_hbm, idx_v, rows_v, pos_v, sem):
    wid = lax.axis_index("s") * _NC + lax.axis_index("c")
    base = wid * _BPW
    l_base = lax.rem(base, _MAX_LEN)

    pltpu.sync_copy(seq_hbm.at[pl.ds(base, _BPW)], idx_v)
    gather = pltpu.async_copy(tok_hbm.at[idx_v], rows_v, sem)
    pltpu.sync_copy(pos_hbm.at[pl.ds(l_base, _BPW)], pos_v)
    gather.wait()

    def add_row(i, carry):
        for j in range(_EMBED_DIM // _L):
            sl = pl.ds(j * _L, _L)
            rows_v[i, sl] = rows_v[i, sl] + pos_v[i, sl]
        return carry

    lax.fori_loop(0, _BPW, add_row, 0)

    pltpu.sync_copy(rows_v, out_hbm.at[pl.ds(base, _BPW)])


@jax.jit
def _embed(seq_flat, tok_table, pos_table):
    mesh = plsc.VectorSubcoreMesh(core_axis_name="c", subcore_axis_name="s")
    f = pl.kernel(
        _embed_body,
        mesh=mesh,
        out_type=jax.ShapeDtypeStruct((_B, _EMBED_DIM), jnp.float32),
        scratch_types=[
            pltpu.VMEM((_BPW,), jnp.int32),
            pltpu.VMEM((_BPW, _EMBED_DIM), jnp.float32),
            pltpu.VMEM((_BPW, _EMBED_DIM), jnp.float32),
            pltpu.SemaphoreType.DMA,
        ],
    )
    return f(seq_flat, tok_table, pos_table)


def kernel(seq, tok_table, pos_table):
    seq_flat = seq.reshape(-1).astype(jnp.int32)
    out = _embed(seq_flat, tok_table, pos_table)
    return out.reshape(_BATCH, _MAX_LEN, _EMBED_DIM)


# speedup vs baseline: 1.3411x; 1.0492x over previous
import functools

import jax
import jax.numpy as jnp
from jax import lax
from jax.experimental import pallas as pl
from jax.experimental.pallas import tpu as pltpu
from jax.experimental.pallas import tpu_sc as plsc

_VOCAB = 100000
_MAX_LEN = 2048
_EMBED_DIM = 128
_BATCH = 4
_B = _BATCH * _MAX_LEN
_NC = 2
_NS = 16
_NW = _NC * _NS
_BPW = _B // _NW
_L = 16


def _embed_body(seq_hbm, tok_hbm, pos_hbm, out_hbm, idx_v, rows_v, sem):
    wid = lax.axis_index("s") * _NC + lax.axis_index("c")
    base = wid * _BPW
    l_base = lax.rem(base, _MAX_LEN)

    pltpu.sync_copy(seq_hbm.at[pl.ds(base, _BPW)], idx_v)
    pltpu.sync_copy(pos_hbm.at[pl.ds(l_base, _BPW)], rows_v)
    pltpu.async_copy(tok_hbm.at[idx_v], rows_v, sem, add=True).wait()
    pltpu.sync_copy(rows_v, out_hbm.at[pl.ds(base, _BPW)])


@jax.jit
def _embed(seq_flat, tok_table, pos_table):
    mesh = plsc.VectorSubcoreMesh(core_axis_name="c", subcore_axis_name="s")
    f = pl.kernel(
        _embed_body,
        mesh=mesh,
        out_type=jax.ShapeDtypeStruct((_B, _EMBED_DIM), jnp.float32),
        scratch_types=[
            pltpu.VMEM((_BPW,), jnp.int32),
            pltpu.VMEM((_BPW, _EMBED_DIM), jnp.float32),
            pltpu.SemaphoreType.DMA,
        ],
    )
    return f(seq_flat, tok_table, pos_table)


def kernel(seq, tok_table, pos_table):
    seq_flat = seq.reshape(-1).astype(jnp.int32)
    out = _embed(seq_flat, tok_table, pos_table)
    return out.reshape(_BATCH, _MAX_LEN, _EMBED_DIM)
